# Initial kernel scaffold; baseline (speedup 1.0000x reference)
import jax
import jax.numpy as jnp
from jax.experimental import pallas as pl


def kernel(x, edge_index, W_in, b_in, W_pred, b_pred):
    raise NotImplementedError("write your pallas kernel here")


# speedup vs baseline: 4.3622x; 4.3622x over previous
import functools

import jax
import jax.numpy as jnp
from jax import lax
from jax.experimental import pallas as pl
from jax.experimental.pallas import tpu as pltpu
from jax.experimental.pallas import tpu_sc as plsc

N = 10000
D = 128
E = 320000

NC = 2
NS = 16
NW = NC * NS
CH = 128
CPW = (-(-E // (CH * NW)) + 7) // 8 * 8
E_PAD = CPW * CH * NW
N_CHUNK_ROWS = E_PAD // CH
R_ACC = 10240
RPT = R_ACC // NS
ROW_BLK = 2000


def _sc_step_body(h_hbm, src_hbm, dst_hbm, p_out, cnt_out,
                  acc, cnt_acc, src_blk, dst_blk, rows, zbuf, zcnt,
                  ones_v, cflush, sem):
    c = lax.axis_index("c")
    s = lax.axis_index("s")
    w = c * NS + s

    def _zb(i, _):
        zbuf[i // 8, pl.ds((i % 8) * 16, 16)] = jnp.zeros((16,), jnp.float32)
        return 0
    lax.fori_loop(0, 128, _zb, 0)

    def _zc(i, _):
        zcnt[pl.ds(i * 16, 16)] = jnp.zeros((16,), jnp.float32)
        return 0
    lax.fori_loop(0, RPT // 16, _zc, 0)

    def _on(i, _):
        ones_v[pl.ds(i * 16, 16)] = jnp.ones((16,), jnp.float32)
        return 0
    lax.fori_loop(0, CH // 16, _on, 0)

    def _za(i, _):
        pltpu.sync_copy(zbuf, acc.at[pl.ds(s * RPT + i * 16, 16)])
        return 0
    lax.fori_loop(0, RPT // 16, _za, 0)
    pltpu.sync_copy(zcnt, cnt_acc.at[pl.ds(s * RPT, RPT)])

    plsc.subcore_barrier()

    base_row = w * CPW
    pltpu.sync_copy(src_hbm.at[pl.ds(base_row, CPW)], src_blk)
    pltpu.sync_copy(dst_hbm.at[pl.ds(base_row, CPW)], dst_blk)

    def _edge(j, _):
        pltpu.async_copy(h_hbm.at[src_blk.at[j]], rows, sem).wait()
        pltpu.sync_copy(rows, acc.at[dst_blk.at[j]], add=True)
        pltpu.sync_copy(ones_v, cnt_acc.at[dst_blk.at[j]], add=True)
        return 0
    lax.fori_loop(0, CPW, _edge, 0)

    plsc.subcore_barrier()

    def _fl(i, _):
        r0 = s * RPT + i * CH
        pltpu.sync_copy(acc.at[pl.ds(r0, CH)], rows)
        pltpu.sync_copy(rows, p_out.at[c].at[pl.ds(r0, CH)])
        return 0
    lax.fori_loop(0, RPT // CH, _fl, 0)

    pltpu.sync_copy(cnt_acc.at[pl.ds(s * RPT, RPT)], cflush)
    pltpu.sync_copy(cflush, cnt_out.at[c].at[pl.ds(s * RPT, RPT)])


_sc_step = pl.kernel(
    _sc_step_body,
    out_type=[jax.ShapeDtypeStruct((NC, R_ACC, D), jnp.float32),
              jax.ShapeDtypeStruct((NC, R_ACC), jnp.float32)],
    mesh=plsc.VectorSubcoreMesh(core_axis_name="c", subcore_axis_name="s"),
    scratch_types=[
        pltpu.VMEM_SHARED((R_ACC, D), jnp.float32),
        pltpu.VMEM_SHARED((R_ACC,), jnp.float32),
        pltpu.VMEM((CPW, CH), jnp.int32),
        pltpu.VMEM((CPW, CH), jnp.int32),
        pltpu.VMEM((CH, D), jnp.float32),
        pltpu.VMEM((16, D), jnp.float32),
        pltpu.VMEM((RPT,), jnp.float32),
        pltpu.VMEM((CH,), jnp.float32),
        pltpu.VMEM((RPT,), jnp.float32),
        pltpu.SemaphoreType.DMA,
    ],
)


def _in_layer_body(x_ref, w_ref, b_ref, o_ref):
    o_ref[...] = jnp.maximum(
        jnp.dot(x_ref[...], w_ref[...], preferred_element_type=jnp.float32)
        + b_ref[...], 0.0)


def _combine_body(h_ref, p0_ref, p1_ref, c0_ref, c1_ref, o_ref):
    inv = 0.5 / jnp.maximum(c0_ref[...] + c1_ref[...], 1.0)
    o_ref[...] = 0.5 * h_ref[...] + inv * (p0_ref[...] + p1_ref[...])


def _finalize_body(h_ref, p0_ref, p1_ref, c0_ref, c1_ref, wp_ref, bp_ref,
                   o_ref, acc_ref):
    i = pl.program_id(0)

    @pl.when(i == 0)
    def _():
        acc_ref[...] = jnp.zeros_like(acc_ref)

    inv = 0.5 / jnp.maximum(c0_ref[...] + c1_ref[...], 1.0)
    h2 = 0.5 * h_ref[...] + inv * (p0_ref[...] + p1_ref[...])
    acc_ref[...] += jnp.sum(h2, axis=0, keepdims=True)

    @pl.when(i == pl.num_programs(0) - 1)
    def _():
        o_ref[...] = (jnp.sum(acc_ref[...] * wp_ref[...], axis=1,
                              keepdims=True) / N + bp_ref[...])


def kernel(x, edge_index, W_in, b_in, W_pred, b_pred):
    src = edge_index[0]
    dst = edge_index[1]
    pad = E_PAD - E
    src_p = jnp.concatenate(
        [src, jnp.zeros((pad,), jnp.int32)]).reshape(N_CHUNK_ROWS, CH)
    dst_p = jnp.concatenate(
        [dst, jnp.full((pad,), N, jnp.int32)]).reshape(N_CHUNK_ROWS, CH)

    h = pl.pallas_call(
        _in_layer_body,
        grid=(N // ROW_BLK,),
        in_specs=[pl.BlockSpec((ROW_BLK, D), lambda i: (i, 0)),
                  pl.BlockSpec((D, D), lambda i: (0, 0)),
                  pl.BlockSpec((1, D), lambda i: (0, 0))],
        out_specs=pl.BlockSpec((ROW_BLK, D), lambda i: (i, 0)),
        out_shape=jax.ShapeDtypeStruct((N, D), jnp.float32),
    )(x, W_in, b_in.reshape(1, D))

    p, cnt = _sc_step(h, src_p, dst_p)
    h = pl.pallas_call(
        _combine_body,
        grid=(N // ROW_BLK,),
        in_specs=[pl.BlockSpec((ROW_BLK, D), lambda i: (i, 0)),
                  pl.BlockSpec((ROW_BLK, D), lambda i: (i, 0)),
                  pl.BlockSpec((ROW_BLK, D), lambda i: (i, 0)),
                  pl.BlockSpec((ROW_BLK, 1), lambda i: (i, 0)),
                  pl.BlockSpec((ROW_BLK, 1), lambda i: (i, 0))],
        out_specs=pl.BlockSpec((ROW_BLK, D), lambda i: (i, 0)),
        out_shape=jax.ShapeDtypeStruct((N, D), jnp.float32),
    )(h, p[0, :N], p[1, :N], cnt[0, :N, None], cnt[1, :N, None])

    p, cnt = _sc_step(h, src_p, dst_p)
    out = pl.pallas_call(
        _finalize_body,
        grid=(N // ROW_BLK,),
        in_specs=[pl.BlockSpec((ROW_BLK, D), lambda i: (i, 0)),
                  pl.BlockSpec((ROW_BLK, D), lambda i: (i, 0)),
                  pl.BlockSpec((ROW_BLK, D), lambda i: (i, 0)),
                  pl.BlockSpec((ROW_BLK, 1), lambda i: (i, 0)),
                  pl.BlockSpec((ROW_BLK, 1), lambda i: (i, 0)),
                  pl.BlockSpec((1, D), lambda i: (0, 0)),
                  pl.BlockSpec((1, 1), lambda i: (0, 0))],
        out_specs=pl.BlockSpec((1, 1), lambda i: (0, 0)),
        out_shape=jax.ShapeDtypeStruct((1, 1), jnp.float32),
        scratch_shapes=[pltpu.VMEM((1, D), jnp.float32)],
    )(h, p[0, :N], p[1, :N], cnt[0, :N, None], cnt[1, :N, None],
      W_pred[:D, 0].reshape(1, D), b_pred.reshape(1, 1))

    return out.reshape(1)


# speedup vs baseline: 4.5000x; 1.0316x over previous
import functools

import jax
import jax.numpy as jnp
from jax import lax
from jax.experimental import pallas as pl
from jax.experimental.pallas import tpu as pltpu
from jax.experimental.pallas import tpu_sc as plsc

N = 10000
D = 128
E = 320000

NC = 2
NS = 16
NW = NC * NS
CH = 128
CPW = (-(-E // (CH * NW)) + 7) // 8 * 8
E_PAD = CPW * CH * NW
N_CHUNK_ROWS = E_PAD // CH
R_ACC = 10240
RPT = R_ACC // NS
NBUF = 2
SLAB = 16
NSLAB = CPW // SLAB
NGRP_S = SLAB // NBUF
FLP = RPT // CH
ROW_BLK = 2000


def _sc_step_body(h_hbm, src_hbm, dst_hbm, p_out, cnt_out,
                  acc, cnt_acc, src_sl, dst_sl, rows, zbuf, zcnt,
                  ones_v, cflush, gsem, ssem, isem):
    c = lax.axis_index("c")
    s = lax.axis_index("s")
    w = c * NS + s

    def _zb(i, _):
        zbuf[i // 8, pl.ds((i % 8) * 16, 16)] = jnp.zeros((16,), jnp.float32)
        return 0
    lax.fori_loop(0, 128, _zb, 0)

    def _zc(i, _):
        zcnt[pl.ds(i * 16, 16)] = jnp.zeros((16,), jnp.float32)
        return 0
    lax.fori_loop(0, RPT // 16, _zc, 0)

    def _on(i, _):
        ones_v[pl.ds(i * 16, 16)] = jnp.ones((16,), jnp.float32)
        return 0
    lax.fori_loop(0, CH // 16, _on, 0)

    def _za(i, _):
        pltpu.sync_copy(zbuf, acc.at[pl.ds(s * RPT + i * 16, 16)])
        return 0
    lax.fori_loop(0, RPT // 16, _za, 0)
    pltpu.sync_copy(zcnt, cnt_acc.at[pl.ds(s * RPT, RPT)])

    plsc.subcore_barrier()

    base_row = w * CPW
    pltpu.sync_copy(src_hbm.at[pl.ds(base_row, SLAB)], src_sl.at[0])
    pltpu.sync_copy(dst_hbm.at[pl.ds(base_row, SLAB)], dst_sl.at[0])

    for b in range(NBUF):
        pltpu.async_copy(h_hbm.at[src_sl.at[0].at[b]], rows[b], gsem[b])

    def _slab(k, _):
        par = lax.rem(k, 2)
        nxt = 1 - par

        @pl.when(k < NSLAB - 1)
        def _():
            r0 = base_row + (k + 1) * SLAB
            pltpu.async_copy(src_hbm.at[pl.ds(r0, SLAB)], src_sl.at[nxt],
                             isem[0])
            pltpu.async_copy(dst_hbm.at[pl.ds(r0, SLAB)], dst_sl.at[nxt],
                             isem[1])

        def _grp(g, _):
            for b in range(NBUF):
                r = g * NBUF + b
                pltpu.make_async_copy(
                    h_hbm.at[src_sl.at[par].at[r]], rows[b], gsem[b]).wait()
                pltpu.async_copy(rows[b], acc.at[dst_sl.at[par].at[r]],
                                 ssem[b], add=True)
                pltpu.async_copy(ones_v, cnt_acc.at[dst_sl.at[par].at[r]],
                                 ssem[b], add=True)
            for b in range(NBUF):
                r = g * NBUF + b
                pltpu.make_async_copy(
                    rows[b], acc.at[dst_sl.at[par].at[r]], ssem[b]).wait()
                pltpu.make_async_copy(
                    ones_v, cnt_acc.at[dst_sl.at[par].at[r]], ssem[b]).wait()

            @pl.when(g < NGRP_S - 1)
            def _():
                for b in range(NBUF):
                    pltpu.async_copy(
                        h_hbm.at[src_sl.at[par].at[(g + 1) * NBUF + b]],
                        rows[b], gsem[b])

            @pl.when((g == NGRP_S - 1) & (k < NSLAB - 1))
            def _():
                r0 = base_row + (k + 1) * SLAB
                pltpu.make_async_copy(src_hbm.at[pl.ds(r0, SLAB)],
                                      src_sl.at[nxt], isem[0]).wait()
                pltpu.make_async_copy(dst_hbm.at[pl.ds(r0, SLAB)],
                                      dst_sl.at[nxt], isem[1]).wait()
                for b in range(NBUF):
                    pltpu.async_copy(h_hbm.at[src_sl.at[nxt].at[b]],
                                     rows[b], gsem[b])
            return 0
        lax.fori_loop(0, NGRP_S, _grp, 0)
        return 0
    lax.fori_loop(0, NSLAB, _slab, 0)

    plsc.subcore_barrier()

    for i in range(FLP):
        b = i % NBUF
        r0 = s * RPT + i * CH
        if i >= NBUF:
            pltpu.make_async_copy(
                rows[b], p_out.at[c].at[pl.ds(s * RPT + (i - NBUF) * CH, CH)],
                gsem[b]).wait()
        pltpu.sync_copy(acc.at[pl.ds(r0, CH)], rows[b])
        pltpu.async_copy(rows[b], p_out.at[c].at[pl.ds(r0, CH)], gsem[b])
    for i in range(max(FLP - NBUF, 0), FLP):
        b = i % NBUF
        r0 = s * RPT + i * CH
        pltpu.make_async_copy(rows[b], p_out.at[c].at[pl.ds(r0, CH)],
                              gsem[b]).wait()

    pltpu.sync_copy(cnt_acc.at[pl.ds(s * RPT, RPT)], cflush)
    pltpu.sync_copy(cflush, cnt_out.at[c].at[pl.ds(s * RPT, RPT)])


_sc_step = pl.kernel(
    _sc_step_body,
    out_type=[jax.ShapeDtypeStruct((NC, R_ACC, D), jnp.float32),
              jax.ShapeDtypeStruct((NC, R_ACC), jnp.float32)],
    mesh=plsc.VectorSubcoreMesh(core_axis_name="c", subcore_axis_name="s"),
    scratch_types=[
        pltpu.VMEM_SHARED((R_ACC, D), jnp.float32),
        pltpu.VMEM_SHARED((R_ACC,), jnp.float32),
        pltpu.VMEM((2, SLAB, CH), jnp.int32),
        pltpu.VMEM((2, SLAB, CH), jnp.int32),
        [pltpu.VMEM((CH, D), jnp.float32) for _ in range(NBUF)],
        pltpu.VMEM((16, D), jnp.float32),
        pltpu.VMEM((RPT,), jnp.float32),
        pltpu.VMEM((CH,), jnp.float32),
        pltpu.VMEM((RPT,), jnp.float32),
        [pltpu.SemaphoreType.DMA for _ in range(NBUF)],
        [pltpu.SemaphoreType.DMA for _ in range(NBUF)],
        [pltpu.SemaphoreType.DMA for _ in range(2)],
    ],
)


def _in_layer_body(x_ref, w_ref, b_ref, o_ref):
    o_ref[...] = jnp.maximum(
        jnp.dot(x_ref[...], w_ref[...], preferred_element_type=jnp.float32)
        + b_ref[...], 0.0)


def _combine_body(h_ref, p0_ref, p1_ref, c0_ref, c1_ref, o_ref):
    inv = 0.5 / jnp.maximum(c0_ref[...] + c1_ref[...], 1.0)
    o_ref[...] = 0.5 * h_ref[...] + inv * (p0_ref[...] + p1_ref[...])


def _finalize_body(h_ref, p0_ref, p1_ref, c0_ref, c1_ref, wp_ref, bp_ref,
                   o_ref, acc_ref):
    i = pl.program_id(0)

    @pl.when(i == 0)
    def _():
        acc_ref[...] = jnp.zeros_like(acc_ref)

    inv = 0.5 / jnp.maximum(c0_ref[...] + c1_ref[...], 1.0)
    h2 = 0.5 * h_ref[...] + inv * (p0_ref[...] + p1_ref[...])
    acc_ref[...] += jnp.sum(h2, axis=0, keepdims=True)

    @pl.when(i == pl.num_programs(0) - 1)
    def _():
        o_ref[...] = (jnp.sum(acc_ref[...] * wp_ref[...], axis=1,
                              keepdims=True) / N + bp_ref[...])


def kernel(x, edge_index, W_in, b_in, W_pred, b_pred):
    src = edge_index[0]
    dst = edge_index[1]
    pad = E_PAD - E
    src_p = jnp.concatenate(
        [src, jnp.zeros((pad,), jnp.int32)]).reshape(N_CHUNK_ROWS, CH)
    dst_p = jnp.concatenate(
        [dst, jnp.full((pad,), N, jnp.int32)]).reshape(N_CHUNK_ROWS, CH)

    h = pl.pallas_call(
        _in_layer_body,
        grid=(N // ROW_BLK,),
        in_specs=[pl.BlockSpec((ROW_BLK, D), lambda i: (i, 0)),
                  pl.BlockSpec((D, D), lambda i: (0, 0)),
                  pl.BlockSpec((1, D), lambda i: (0, 0))],
        out_specs=pl.BlockSpec((ROW_BLK, D), lambda i: (i, 0)),
        out_shape=jax.ShapeDtypeStruct((N, D), jnp.float32),
    )(x, W_in, b_in.reshape(1, D))

    p, cnt = _sc_step(h, src_p, dst_p)
    h = pl.pallas_call(
        _combine_body,
        grid=(N // ROW_BLK,),
        in_specs=[pl.BlockSpec((ROW_BLK, D), lambda i: (i, 0)),
                  pl.BlockSpec((ROW_BLK, D), lambda i: (i, 0)),
                  pl.BlockSpec((ROW_BLK, D), lambda i: (i, 0)),
                  pl.BlockSpec((ROW_BLK, 1), lambda i: (i, 0)),
                  pl.BlockSpec((ROW_BLK, 1), lambda i: (i, 0))],
        out_specs=pl.BlockSpec((ROW_BLK, D), lambda i: (i, 0)),
        out_shape=jax.ShapeDtypeStruct((N, D), jnp.float32),
    )(h, p[0, :N], p[1, :N], cnt[0, :N, None], cnt[1, :N, None])

    p, cnt = _sc_step(h, src_p, dst_p)
    out = pl.pallas_call(
        _finalize_body,
        grid=(N // ROW_BLK,),
        in_specs=[pl.BlockSpec((ROW_BLK, D), lambda i: (i, 0)),
                  pl.BlockSpec((ROW_BLK, D), lambda i: (i, 0)),
                  pl.BlockSpec((ROW_BLK, D), lambda i: (i, 0)),
                  pl.BlockSpec((ROW_BLK, 1), lambda i: (i, 0)),
                  pl.BlockSpec((ROW_BLK, 1), lambda i: (i, 0)),
                  pl.BlockSpec((1, D), lambda i: (0, 0)),
                  pl.BlockSpec((1, 1), lambda i: (0, 0))],
        out_specs=pl.BlockSpec((1, 1), lambda i: (0, 0)),
        out_shape=jax.ShapeDtypeStruct((1, 1), jnp.float32),
        scratch_shapes=[pltpu.VMEM((1, D), jnp.float32)],
    )(h, p[0, :N], p[1, :N], cnt[0, :N, None], cnt[1, :N, None],
      W_pred[:D, 0].reshape(1, D), b_pred.reshape(1, 1))

    return out.reshape(1)


# speedup vs baseline: 13.0860x; 2.9080x over previous
import functools

import jax
import jax.numpy as jnp
from jax import lax
from jax.experimental import pallas as pl
from jax.experimental.pallas import tpu as pltpu
from jax.experimental.pallas import tpu_sc as plsc

N = 10000
D = 128
E = 320000

NC = 2
NS = 16
NW = NC * NS
CH = 128
CPW = (-(-E // (CH * NW)) + 7) // 8 * 8
E_PAD = CPW * CH * NW
N_CHUNK_ROWS = E_PAD // CH
R_ACC = 10240
RPT = R_ACC // NS
NBUF = 2
SLAB = 16
NSLAB = CPW // SLAB
NGRP_S = SLAB // NBUF
FLP = RPT // CH
ROW_BLK = 2000


def _sc_step_body(h_hbm, src_hbm, dst_hbm, p_out, cnt_out,
                  acc, cnt_acc, src_sl, dst_sl, rows, zbuf, zcnt,
                  ones_v, cflush, gsem, ssem, isem):
    c = lax.axis_index("c")
    s = lax.axis_index("s")
    w = c * NS + s

    def _zb(i, _):
        zbuf[i // 8, pl.ds((i % 8) * 16, 16)] = jnp.zeros((16,), jnp.float32)
        return 0
    lax.fori_loop(0, 128, _zb, 0)

    def _zc(i, _):
        zcnt[pl.ds(i * 16, 16)] = jnp.zeros((16,), jnp.float32)
        return 0
    lax.fori_loop(0, RPT // 16, _zc, 0)

    def _on(i, _):
        ones_v[pl.ds(i * 16, 16)] = jnp.ones((16,), jnp.float32)
        return 0
    lax.fori_loop(0, CH // 16, _on, 0)

    def _za(i, _):
        pltpu.sync_copy(zbuf, acc.at[pl.ds(s * RPT + i * 16, 16)])
        return 0
    lax.fori_loop(0, RPT // 16, _za, 0)
    pltpu.sync_copy(zcnt, cnt_acc.at[pl.ds(s * RPT, RPT)])

    plsc.subcore_barrier()

    base_row = w * CPW
    pltpu.sync_copy(src_hbm.at[pl.ds(base_row, SLAB)], src_sl.at[0])
    pltpu.sync_copy(dst_hbm.at[pl.ds(base_row, SLAB)], dst_sl.at[0])

    for b in range(NBUF):
        pltpu.async_copy(h_hbm.at[src_sl.at[0].at[b]], rows[b], gsem[b])

    def _slab(k, _):
        par = lax.rem(k, 2)
        nxt = 1 - par

        @pl.when(k < NSLAB - 1)
        def _():
            r0 = base_row + (k + 1) * SLAB
            pltpu.async_copy(src_hbm.at[pl.ds(r0, SLAB)], src_sl.at[nxt],
                             isem[0])
            pltpu.async_copy(dst_hbm.at[pl.ds(r0, SLAB)], dst_sl.at[nxt],
                             isem[1])

        def _grp(g, _):
            for b in range(NBUF):
                r = g * NBUF + b
                pltpu.make_async_copy(
                    h_hbm.at[src_sl.at[par].at[r]], rows[b], gsem[b]).wait()
                pltpu.async_copy(rows[b], acc.at[dst_sl.at[par].at[r]],
                                 ssem[b], add=True)
                pltpu.async_copy(ones_v, cnt_acc.at[dst_sl.at[par].at[r]],
                                 ssem[b], add=True)
            for b in range(NBUF):
                r = g * NBUF + b
                pltpu.make_async_copy(
                    rows[b], acc.at[dst_sl.at[par].at[r]], ssem[b]).wait()
                pltpu.make_async_copy(
                    ones_v, cnt_acc.at[dst_sl.at[par].at[r]], ssem[b]).wait()

            @pl.when(g < NGRP_S - 1)
            def _():
                for b in range(NBUF):
                    pltpu.async_copy(
                        h_hbm.at[src_sl.at[par].at[(g + 1) * NBUF + b]],
                        rows[b], gsem[b])

            @pl.when((g == NGRP_S - 1) & (k < NSLAB - 1))
            def _():
                r0 = base_row + (k + 1) * SLAB
                pltpu.make_async_copy(src_hbm.at[pl.ds(r0, SLAB)],
                                      src_sl.at[nxt], isem[0]).wait()
                pltpu.make_async_copy(dst_hbm.at[pl.ds(r0, SLAB)],
                                      dst_sl.at[nxt], isem[1]).wait()
                for b in range(NBUF):
                    pltpu.async_copy(h_hbm.at[src_sl.at[nxt].at[b]],
                                     rows[b], gsem[b])
            return 0
        lax.fori_loop(0, NGRP_S, _grp, 0)
        return 0
    lax.fori_loop(0, NSLAB, _slab, 0)

    plsc.subcore_barrier()

    for i in range(FLP):
        b = i % NBUF
        r0 = s * RPT + i * CH
        if i >= NBUF:
            pltpu.make_async_copy(
                rows[b], p_out.at[c].at[pl.ds(s * RPT + (i - NBUF) * CH, CH)],
                gsem[b]).wait()
        pltpu.sync_copy(acc.at[pl.ds(r0, CH)], rows[b])
        pltpu.async_copy(rows[b], p_out.at[c].at[pl.ds(r0, CH)], gsem[b])
    for i in range(max(FLP - NBUF, 0), FLP):
        b = i % NBUF
        r0 = s * RPT + i * CH
        pltpu.make_async_copy(rows[b], p_out.at[c].at[pl.ds(r0, CH)],
                              gsem[b]).wait()

    pltpu.sync_copy(cnt_acc.at[pl.ds(s * RPT, RPT)], cflush)
    pltpu.sync_copy(cflush, cnt_out.at[c].at[pl.ds(s * RPT, RPT)])


_sc_step = pl.kernel(
    _sc_step_body,
    out_type=[jax.ShapeDtypeStruct((NC, R_ACC, D), jnp.float32),
              jax.ShapeDtypeStruct((NC, R_ACC), jnp.float32)],
    mesh=plsc.VectorSubcoreMesh(core_axis_name="c", subcore_axis_name="s"),
    scratch_types=[
        pltpu.VMEM_SHARED((R_ACC, D), jnp.float32),
        pltpu.VMEM_SHARED((R_ACC,), jnp.float32),
        pltpu.VMEM((2, SLAB, CH), jnp.int32),
        pltpu.VMEM((2, SLAB, CH), jnp.int32),
        [pltpu.VMEM((CH, D), jnp.float32) for _ in range(NBUF)],
        pltpu.VMEM((16, D), jnp.float32),
        pltpu.VMEM((RPT,), jnp.float32),
        pltpu.VMEM((CH,), jnp.float32),
        pltpu.VMEM((RPT,), jnp.float32),
        [pltpu.SemaphoreType.DMA for _ in range(NBUF)],
        [pltpu.SemaphoreType.DMA for _ in range(NBUF)],
        [pltpu.SemaphoreType.DMA for _ in range(2)],
    ],
)


def _in_layer_body(x_ref, w_ref, b_ref, o_ref):
    o_ref[...] = jnp.maximum(
        jnp.dot(x_ref[...], w_ref[...], preferred_element_type=jnp.float32)
        + b_ref[...], 0.0)


def _combine_body(h_ref, p0_ref, p1_ref, c0_ref, c1_ref, o_ref):
    inv = 0.5 / jnp.maximum(c0_ref[...] + c1_ref[...], 1.0)
    o_ref[...] = 0.5 * h_ref[...] + inv * (p0_ref[...] + p1_ref[...])


def _finalize_body(h_ref, p0_ref, p1_ref, c0_ref, c1_ref, wp_ref, bp_ref,
                   o_ref, acc_ref):
    i = pl.program_id(0)

    @pl.when(i == 0)
    def _():
        acc_ref[...] = jnp.zeros_like(acc_ref)

    inv = 0.5 / jnp.maximum(c0_ref[...] + c1_ref[...], 1.0)
    h2 = 0.5 * h_ref[...] + inv * (p0_ref[...] + p1_ref[...])
    acc_ref[...] += jnp.sum(h2, axis=0, keepdims=True)

    @pl.when(i == pl.num_programs(0) - 1)
    def _():
        o_ref[...] = (jnp.sum(acc_ref[...] * wp_ref[...], axis=1,
                              keepdims=True) / N + bp_ref[...])


def kernel(x, edge_index, W_in, b_in, W_pred, b_pred):
    src = edge_index[0]
    dst = edge_index[1]
    pad = E_PAD - E
    pad_ids = jnp.arange(pad, dtype=jnp.int32)
    src_p = jnp.concatenate(
        [src, pad_ids % N]).reshape(N_CHUNK_ROWS, CH)
    dst_p = jnp.concatenate(
        [dst, N + pad_ids % (R_ACC - N)]).reshape(N_CHUNK_ROWS, CH)

    h = pl.pallas_call(
        _in_layer_body,
        grid=(N // ROW_BLK,),
        in_specs=[pl.BlockSpec((ROW_BLK, D), lambda i: (i, 0)),
                  pl.BlockSpec((D, D), lambda i: (0, 0)),
                  pl.BlockSpec((1, D), lambda i: (0, 0))],
        out_specs=pl.BlockSpec((ROW_BLK, D), lambda i: (i, 0)),
        out_shape=jax.ShapeDtypeStruct((N, D), jnp.float32),
    )(x, W_in, b_in.reshape(1, D))

    p, cnt = _sc_step(h, src_p, dst_p)
    h = pl.pallas_call(
        _combine_body,
        grid=(N // ROW_BLK,),
        in_specs=[pl.BlockSpec((ROW_BLK, D), lambda i: (i, 0)),
                  pl.BlockSpec((ROW_BLK, D), lambda i: (i, 0)),
                  pl.BlockSpec((ROW_BLK, D), lambda i: (i, 0)),
                  pl.BlockSpec((ROW_BLK, 1), lambda i: (i, 0)),
                  pl.BlockSpec((ROW_BLK, 1), lambda i: (i, 0))],
        out_specs=pl.BlockSpec((ROW_BLK, D), lambda i: (i, 0)),
        out_shape=jax.ShapeDtypeStruct((N, D), jnp.float32),
    )(h, p[0, :N], p[1, :N], cnt[0, :N, None], cnt[1, :N, None])

    p, cnt = _sc_step(h, src_p, dst_p)
    out = pl.pallas_call(
        _finalize_body,
        grid=(N // ROW_BLK,),
        in_specs=[pl.BlockSpec((ROW_BLK, D), lambda i: (i, 0)),
                  pl.BlockSpec((ROW_BLK, D), lambda i: (i, 0)),
                  pl.BlockSpec((ROW_BLK, D), lambda i: (i, 0)),
                  pl.BlockSpec((ROW_BLK, 1), lambda i: (i, 0)),
                  pl.BlockSpec((ROW_BLK, 1), lambda i: (i, 0)),
                  pl.BlockSpec((1, D), lambda i: (0, 0)),
                  pl.BlockSpec((1, 1), lambda i: (0, 0))],
        out_specs=pl.BlockSpec((1, 1), lambda i: (0, 0)),
        out_shape=jax.ShapeDtypeStruct((1, 1), jnp.float32),
        scratch_shapes=[pltpu.VMEM((1, D), jnp.float32)],
    )(h, p[0, :N], p[1, :N], cnt[0, :N, None], cnt[1, :N, None],
      W_pred[:D, 0].reshape(1, D), b_pred.reshape(1, 1))

    return out.reshape(1)


# speedup vs baseline: 13.9808x; 1.0684x over previous
import functools

import jax
import jax.numpy as jnp
from jax import lax
from jax.experimental import pallas as pl
from jax.experimental.pallas import tpu as pltpu
from jax.experimental.pallas import tpu_sc as plsc

N = 10000
D = 128
E = 320000

NC = 2
NS = 16
NW = NC * NS
CH = 128
CPW = (-(-E // (CH * NW)) + 7) // 8 * 8
E_PAD = CPW * CH * NW
N_CHUNK_ROWS = E_PAD // CH
R_ACC = 10240
RPT = R_ACC // NS
NBUF = 2
SLAB = 16
NSLAB = CPW // SLAB
NGRP_S = SLAB // NBUF
FLP = RPT // CH
ROW_BLK = 2000


def _sc_step_body(with_counts, h_hbm, src_hbm, dst_hbm, p_out, cnt_out,
                  acc, cnt_acc, src_sl, dst_sl, rows, zbuf, zcnt,
                  ones_v, cflush, gsem, ssem, isem):
    c = lax.axis_index("c")
    s = lax.axis_index("s")
    w = c * NS + s

    def _zb(i, _):
        zbuf[i // 8, pl.ds((i % 8) * 16, 16)] = jnp.zeros((16,), jnp.float32)
        return 0
    lax.fori_loop(0, 128, _zb, 0)

    if with_counts:
        def _zc(i, _):
            zcnt[pl.ds(i * 16, 16)] = jnp.zeros((16,), jnp.float32)
            return 0
        lax.fori_loop(0, RPT // 16, _zc, 0)

        def _on(i, _):
            ones_v[pl.ds(i * 16, 16)] = jnp.ones((16,), jnp.float32)
            return 0
        lax.fori_loop(0, CH // 16, _on, 0)

    def _za(i, _):
        pltpu.sync_copy(zbuf, acc.at[pl.ds(s * RPT + i * 16, 16)])
        return 0
    lax.fori_loop(0, RPT // 16, _za, 0)
    if with_counts:
        pltpu.sync_copy(zcnt, cnt_acc.at[pl.ds(s * RPT, RPT)])

    plsc.subcore_barrier()

    base_row = w * CPW
    pltpu.sync_copy(src_hbm.at[pl.ds(base_row, SLAB)], src_sl.at[0])
    pltpu.sync_copy(dst_hbm.at[pl.ds(base_row, SLAB)], dst_sl.at[0])

    for b in range(NBUF):
        pltpu.async_copy(h_hbm.at[src_sl.at[0].at[b]], rows[b], gsem[b])

    def _slab(k, _):
        par = lax.rem(k, 2)
        nxt = 1 - par

        @pl.when(k < NSLAB - 1)
        def _():
            r0 = base_row + (k + 1) * SLAB
            pltpu.async_copy(src_hbm.at[pl.ds(r0, SLAB)], src_sl.at[nxt],
                             isem[0])
            pltpu.async_copy(dst_hbm.at[pl.ds(r0, SLAB)], dst_sl.at[nxt],
                             isem[1])

        def _grp(g, _):
            for b in range(NBUF):
                r = g * NBUF + b
                pltpu.make_async_copy(
                    h_hbm.at[src_sl.at[par].at[r]], rows[b], gsem[b]).wait()
                pltpu.async_copy(rows[b], acc.at[dst_sl.at[par].at[r]],
                                 ssem[b], add=True)
                if with_counts:
                    pltpu.async_copy(ones_v, cnt_acc.at[dst_sl.at[par].at[r]],
                                     ssem[b], add=True)
            for b in range(NBUF):
                r = g * NBUF + b
                pltpu.make_async_copy(
                    rows[b], acc.at[dst_sl.at[par].at[r]], ssem[b]).wait()
                if with_counts:
                    pltpu.make_async_copy(
                        ones_v, cnt_acc.at[dst_sl.at[par].at[r]],
                        ssem[b]).wait()

            @pl.when(g < NGRP_S - 1)
            def _():
                for b in range(NBUF):
                    pltpu.async_copy(
                        h_hbm.at[src_sl.at[par].at[(g + 1) * NBUF + b]],
                        rows[b], gsem[b])

            @pl.when((g == NGRP_S - 1) & (k < NSLAB - 1))
            def _():
                r0 = base_row + (k + 1) * SLAB
                pltpu.make_async_copy(src_hbm.at[pl.ds(r0, SLAB)],
                                      src_sl.at[nxt], isem[0]).wait()
                pltpu.make_async_copy(dst_hbm.at[pl.ds(r0, SLAB)],
                                      dst_sl.at[nxt], isem[1]).wait()
                for b in range(NBUF):
                    pltpu.async_copy(h_hbm.at[src_sl.at[nxt].at[b]],
                                     rows[b], gsem[b])
            return 0
        lax.fori_loop(0, NGRP_S, _grp, 0)
        return 0
    lax.fori_loop(0, NSLAB, _slab, 0)

    plsc.subcore_barrier()

    for i in range(FLP):
        b = i % NBUF
        r0 = s * RPT + i * CH
        if i >= NBUF:
            pltpu.make_async_copy(
                rows[b], p_out.at[c].at[pl.ds(s * RPT + (i - NBUF) * CH, CH)],
                gsem[b]).wait()
        pltpu.sync_copy(acc.at[pl.ds(r0, CH)], rows[b])
        pltpu.async_copy(rows[b], p_out.at[c].at[pl.ds(r0, CH)], gsem[b])
    for i in range(max(FLP - NBUF, 0), FLP):
        b = i % NBUF
        r0 = s * RPT + i * CH
        pltpu.make_async_copy(rows[b], p_out.at[c].at[pl.ds(r0, CH)],
                              gsem[b]).wait()

    if with_counts:
        pltpu.sync_copy(cnt_acc.at[pl.ds(s * RPT, RPT)], cflush)
        pltpu.sync_copy(cflush, cnt_out.at[c].at[pl.ds(s * RPT, RPT)])


def _make_sc_step(with_counts):
    return pl.kernel(
        functools.partial(_sc_step_body, with_counts),
        out_type=[jax.ShapeDtypeStruct((NC, R_ACC, D), jnp.float32),
                  jax.ShapeDtypeStruct((NC, R_ACC), jnp.float32)],
        mesh=plsc.VectorSubcoreMesh(core_axis_name="c", subcore_axis_name="s"),
        scratch_types=[
            pltpu.VMEM_SHARED((R_ACC, D), jnp.float32),
            pltpu.VMEM_SHARED((R_ACC,), jnp.float32),
            pltpu.VMEM((2, SLAB, CH), jnp.int32),
            pltpu.VMEM((2, SLAB, CH), jnp.int32),
            [pltpu.VMEM((CH, D), jnp.float32) for _ in range(NBUF)],
            pltpu.VMEM((16, D), jnp.float32),
            pltpu.VMEM((RPT,), jnp.float32),
            pltpu.VMEM((CH,), jnp.float32),
            pltpu.VMEM((RPT,), jnp.float32),
            [pltpu.SemaphoreType.DMA for _ in range(NBUF)],
            [pltpu.SemaphoreType.DMA for _ in range(NBUF)],
            [pltpu.SemaphoreType.DMA for _ in range(2)],
        ],
    )


_sc_step_counts = _make_sc_step(True)
_sc_step_nocnt = _make_sc_step(False)


def _in_layer_body(x_ref, w_ref, b_ref, o_ref):
    o_ref[...] = jnp.maximum(
        jnp.dot(x_ref[...], w_ref[...], preferred_element_type=jnp.float32)
        + b_ref[...], 0.0)


def _combine_body(h_ref, p0_ref, p1_ref, c0_ref, c1_ref, o_ref):
    inv = 0.5 / jnp.maximum(c0_ref[0] + c1_ref[0], 1.0)
    o_ref[...] = 0.5 * h_ref[...] + inv * (p0_ref[0] + p1_ref[0])


def _finalize_body(h_ref, p0_ref, p1_ref, c0_ref, c1_ref, wp_ref, bp_ref,
                   o_ref, acc_ref):
    i = pl.program_id(0)

    @pl.when(i == 0)
    def _():
        acc_ref[...] = jnp.zeros_like(acc_ref)

    inv = 0.5 / jnp.maximum(c0_ref[0] + c1_ref[0], 1.0)
    h2 = 0.5 * h_ref[...] + inv * (p0_ref[0] + p1_ref[0])
    acc_ref[...] += jnp.sum(h2, axis=0, keepdims=True)

    @pl.when(i == pl.num_programs(0) - 1)
    def _():
        o_ref[...] = (jnp.sum(acc_ref[...] * wp_ref[...], axis=1,
                              keepdims=True) / N + bp_ref[...])


def kernel(x, edge_index, W_in, b_in, W_pred, b_pred):
    src = edge_index[0]
    dst = edge_index[1]
    pad = E_PAD - E
    pad_ids = jnp.arange(pad, dtype=jnp.int32)
    src_p = jnp.concatenate(
        [src, pad_ids % N]).reshape(N_CHUNK_ROWS, CH)
    dst_p = jnp.concatenate(
        [dst, N + pad_ids % (R_ACC - N)]).reshape(N_CHUNK_ROWS, CH)

    h = pl.pallas_call(
        _in_layer_body,
        grid=(N // ROW_BLK,),
        in_specs=[pl.BlockSpec((ROW_BLK, D), lambda i: (i, 0)),
                  pl.BlockSpec((D, D), lambda i: (0, 0)),
                  pl.BlockSpec((1, D), lambda i: (0, 0))],
        out_specs=pl.BlockSpec((ROW_BLK, D), lambda i: (i, 0)),
        out_shape=jax.ShapeDtypeStruct((N, D), jnp.float32),
    )(x, W_in, b_in.reshape(1, D))

    p, cnt = _sc_step_counts(h, src_p, dst_p)
    cnt3 = cnt[:, :, None]
    h = pl.pallas_call(
        _combine_body,
        grid=(N // ROW_BLK,),
        in_specs=[pl.BlockSpec((ROW_BLK, D), lambda i: (i, 0)),
                  pl.BlockSpec((1, ROW_BLK, D), lambda i: (0, i, 0)),
                  pl.BlockSpec((1, ROW_BLK, D), lambda i: (1, i, 0)),
                  pl.BlockSpec((1, ROW_BLK, 1), lambda i: (0, i, 0)),
                  pl.BlockSpec((1, ROW_BLK, 1), lambda i: (1, i, 0))],
        out_specs=pl.BlockSpec((ROW_BLK, D), lambda i: (i, 0)),
        out_shape=jax.ShapeDtypeStruct((N, D), jnp.float32),
    )(h, p, p, cnt3, cnt3)

    p, _ = _sc_step_nocnt(h, src_p, dst_p)
    out = pl.pallas_call(
        _finalize_body,
        grid=(N // ROW_BLK,),
        in_specs=[pl.BlockSpec((ROW_BLK, D), lambda i: (i, 0)),
                  pl.BlockSpec((1, ROW_BLK, D), lambda i: (0, i, 0)),
                  pl.BlockSpec((1, ROW_BLK, D), lambda i: (1, i, 0)),
                  pl.BlockSpec((1, ROW_BLK, 1), lambda i: (0, i, 0)),
                  pl.BlockSpec((1, ROW_BLK, 1), lambda i: (1, i, 0)),
                  pl.BlockSpec((1, D), lambda i: (0, 0)),
                  pl.BlockSpec((1, 1), lambda i: (0, 0))],
        out_specs=pl.BlockSpec((1, 1), lambda i: (0, 0)),
        out_shape=jax.ShapeDtypeStruct((1, 1), jnp.float32),
        scratch_shapes=[pltpu.VMEM((1, D), jnp.float32)],
    )(h, p, p, cnt3, cnt3,
      W_pred[:D, 0].reshape(1, D), b_pred.reshape(1, 1))

    return out.reshape(1)


# speedup vs baseline: 14.2222x; 1.0173x over previous
import functools

import jax
import jax.numpy as jnp
from jax import lax
from jax.experimental import pallas as pl
from jax.experimental.pallas import tpu as pltpu
from jax.experimental.pallas import tpu_sc as plsc

N = 10000
D = 128
E = 320000

NC = 2
NS = 16
NW = NC * NS
CH = 128
CPW = (-(-E // (CH * NW)) + 7) // 8 * 8
E_PAD = CPW * CH * NW
N_CHUNK_ROWS = E_PAD // CH
R_ACC = 10240
RPT = R_ACC // NS
NBUF = 2
SLAB = 16
NSLAB = CPW // SLAB
NGRP_S = SLAB // NBUF
FLP = RPT // CH
ZROWS = 32
ROW_BLK = 2000


def _sc_step_body(with_counts, h_hbm, src_hbm, dst_hbm, p_out, cnt_out,
                  acc, cnt_acc, src_sl, dst_sl, rows, zbuf, zcnt,
                  ones_v, cflush, gsem, ssem, isem):
    c = lax.axis_index("c")
    s = lax.axis_index("s")
    w = c * NS + s

    def _zb(i, _):
        zbuf[i // 8, pl.ds((i % 8) * 16, 16)] = jnp.zeros((16,), jnp.float32)
        return 0
    lax.fori_loop(0, ZROWS * 8, _zb, 0)

    if with_counts:
        def _zc(i, _):
            zcnt[pl.ds(i * 16, 16)] = jnp.zeros((16,), jnp.float32)
            return 0
        lax.fori_loop(0, RPT // 16, _zc, 0)

        def _on(i, _):
            ones_v[pl.ds(i * 16, 16)] = jnp.ones((16,), jnp.float32)
            return 0
        lax.fori_loop(0, CH // 16, _on, 0)

    for i in range(RPT // ZROWS):
        pltpu.async_copy(zbuf, acc.at[pl.ds(s * RPT + i * ZROWS, ZROWS)],
                         ssem[i % NBUF])
    if with_counts:
        pltpu.sync_copy(zcnt, cnt_acc.at[pl.ds(s * RPT, RPT)])

    base_row = w * CPW
    pltpu.sync_copy(src_hbm.at[pl.ds(base_row, SLAB)], src_sl.at[0])
    pltpu.sync_copy(dst_hbm.at[pl.ds(base_row, SLAB)], dst_sl.at[0])

    for b in range(NBUF):
        pltpu.async_copy(h_hbm.at[src_sl.at[0].at[b]], rows[b], gsem[b])

    for i in range(RPT // ZROWS):
        pltpu.make_async_copy(
            zbuf, acc.at[pl.ds(s * RPT + i * ZROWS, ZROWS)],
            ssem[i % NBUF]).wait()

    plsc.subcore_barrier()

    def _slab(k, _):
        par = lax.rem(k, 2)
        nxt = 1 - par

        @pl.when(k < NSLAB - 1)
        def _():
            r0 = base_row + (k + 1) * SLAB
            pltpu.async_copy(src_hbm.at[pl.ds(r0, SLAB)], src_sl.at[nxt],
                             isem[0])
            pltpu.async_copy(dst_hbm.at[pl.ds(r0, SLAB)], dst_sl.at[nxt],
                             isem[1])

        def _grp(g, _):
            for b in range(NBUF):
                r = g * NBUF + b
                pltpu.make_async_copy(
                    h_hbm.at[src_sl.at[par].at[r]], rows[b], gsem[b]).wait()
                pltpu.async_copy(rows[b], acc.at[dst_sl.at[par].at[r]],
                                 ssem[b], add=True)
                if with_counts:
                    pltpu.async_copy(ones_v, cnt_acc.at[dst_sl.at[par].at[r]],
                                     ssem[b], add=True)
            for b in range(NBUF):
                r = g * NBUF + b
                pltpu.make_async_copy(
                    rows[b], acc.at[dst_sl.at[par].at[r]], ssem[b]).wait()
                if with_counts:
                    pltpu.make_async_copy(
                        ones_v, cnt_acc.at[dst_sl.at[par].at[r]],
                        ssem[b]).wait()

            @pl.when(g < NGRP_S - 1)
            def _():
                for b in range(NBUF):
                    pltpu.async_copy(
                        h_hbm.at[src_sl.at[par].at[(g + 1) * NBUF + b]],
                        rows[b], gsem[b])

            @pl.when((g == NGRP_S - 1) & (k < NSLAB - 1))
            def _():
                r0 = base_row + (k + 1) * SLAB
                pltpu.make_async_copy(src_hbm.at[pl.ds(r0, SLAB)],
                                      src_sl.at[nxt], isem[0]).wait()
                pltpu.make_async_copy(dst_hbm.at[pl.ds(r0, SLAB)],
                                      dst_sl.at[nxt], isem[1]).wait()
                for b in range(NBUF):
                    pltpu.async_copy(h_hbm.at[src_sl.at[nxt].at[b]],
                                     rows[b], gsem[b])
            return 0
        lax.fori_loop(0, NGRP_S, _grp, 0)
        return 0
    lax.fori_loop(0, NSLAB, _slab, 0)

    plsc.subcore_barrier()

    for i in range(FLP):
        b = i % NBUF
        r0 = s * RPT + i * CH
        if i >= NBUF:
            pltpu.make_async_copy(
                rows[b], p_out.at[c].at[pl.ds(s * RPT + (i - NBUF) * CH, CH)],
                gsem[b]).wait()
        pltpu.sync_copy(acc.at[pl.ds(r0, CH)], rows[b])
        pltpu.async_copy(rows[b], p_out.at[c].at[pl.ds(r0, CH)], gsem[b])
    for i in range(max(FLP - NBUF, 0), FLP):
        b = i % NBUF
        r0 = s * RPT + i * CH
        pltpu.make_async_copy(rows[b], p_out.at[c].at[pl.ds(r0, CH)],
                              gsem[b]).wait()

    if with_counts:
        pltpu.sync_copy(cnt_acc.at[pl.ds(s * RPT, RPT)], cflush)
        pltpu.sync_copy(cflush, cnt_out.at[c].at[pl.ds(s * RPT, RPT)])


def _make_sc_step(with_counts):
    return pl.kernel(
        functools.partial(_sc_step_body, with_counts),
        out_type=[jax.ShapeDtypeStruct((NC, R_ACC, D), jnp.float32),
                  jax.ShapeDtypeStruct((NC, R_ACC), jnp.float32)],
        mesh=plsc.VectorSubcoreMesh(core_axis_name="c", subcore_axis_name="s"),
        scratch_types=[
            pltpu.VMEM_SHARED((R_ACC, D), jnp.float32),
            pltpu.VMEM_SHARED((R_ACC,), jnp.float32),
            pltpu.VMEM((2, SLAB, CH), jnp.int32),
            pltpu.VMEM((2, SLAB, CH), jnp.int32),
            [pltpu.VMEM((CH, D), jnp.float32) for _ in range(NBUF)],
            pltpu.VMEM((ZROWS, D), jnp.float32),
            pltpu.VMEM((RPT,), jnp.float32),
            pltpu.VMEM((CH,), jnp.float32),
            pltpu.VMEM((RPT,), jnp.float32),
            [pltpu.SemaphoreType.DMA for _ in range(NBUF)],
            [pltpu.SemaphoreType.DMA for _ in range(NBUF)],
            [pltpu.SemaphoreType.DMA for _ in range(2)],
        ],
    )


_sc_step_counts = _make_sc_step(True)
_sc_step_nocnt = _make_sc_step(False)


def _in_layer_body(x_ref, w_ref, b_ref, o_ref):
    o_ref[...] = jnp.maximum(
        jnp.dot(x_ref[...], w_ref[...], preferred_element_type=jnp.float32)
        + b_ref[...], 0.0)


def _combine_body(h_ref, p0_ref, p1_ref, c0_ref, c1_ref, o_ref):
    inv = 0.5 / jnp.maximum(c0_ref[0] + c1_ref[0], 1.0)
    o_ref[...] = 0.5 * h_ref[...] + inv * (p0_ref[0] + p1_ref[0])


def _finalize_body(h_ref, p0_ref, p1_ref, c0_ref, c1_ref, wp_ref, bp_ref,
                   o_ref, acc_ref):
    i = pl.program_id(0)

    @pl.when(i == 0)
    def _():
        acc_ref[...] = jnp.zeros_like(acc_ref)

    inv = 0.5 / jnp.maximum(c0_ref[0] + c1_ref[0], 1.0)
    h2 = 0.5 * h_ref[...] + inv * (p0_ref[0] + p1_ref[0])
    acc_ref[...] += jnp.sum(h2, axis=0, keepdims=True)

    @pl.when(i == pl.num_programs(0) - 1)
    def _():
        o_ref[...] = (jnp.sum(acc_ref[...] * wp_ref[...], axis=1,
                              keepdims=True) / N + bp_ref[...])


def kernel(x, edge_index, W_in, b_in, W_pred, b_pred):
    src = edge_index[0]
    dst = edge_index[1]
    pad = E_PAD - E
    pad_ids = jnp.arange(pad, dtype=jnp.int32)
    src_p = jnp.concatenate(
        [src, pad_ids % N]).reshape(N_CHUNK_ROWS, CH)
    dst_p = jnp.concatenate(
        [dst, N + pad_ids % (R_ACC - N)]).reshape(N_CHUNK_ROWS, CH)

    h = pl.pallas_call(
        _in_layer_body,
        grid=(N // ROW_BLK,),
        in_specs=[pl.BlockSpec((ROW_BLK, D), lambda i: (i, 0)),
                  pl.BlockSpec((D, D), lambda i: (0, 0)),
                  pl.BlockSpec((1, D), lambda i: (0, 0))],
        out_specs=pl.BlockSpec((ROW_BLK, D), lambda i: (i, 0)),
        out_shape=jax.ShapeDtypeStruct((N, D), jnp.float32),
    )(x, W_in, b_in.reshape(1, D))

    p, cnt = _sc_step_counts(h, src_p, dst_p)
    cnt3 = cnt[:, :, None]
    h = pl.pallas_call(
        _combine_body,
        grid=(N // ROW_BLK,),
        in_specs=[pl.BlockSpec((ROW_BLK, D), lambda i: (i, 0)),
                  pl.BlockSpec((1, ROW_BLK, D), lambda i: (0, i, 0)),
                  pl.BlockSpec((1, ROW_BLK, D), lambda i: (1, i, 0)),
                  pl.BlockSpec((1, ROW_BLK, 1), lambda i: (0, i, 0)),
                  pl.BlockSpec((1, ROW_BLK, 1), lambda i: (1, i, 0))],
        out_specs=pl.BlockSpec((ROW_BLK, D), lambda i: (i, 0)),
        out_shape=jax.ShapeDtypeStruct((N, D), jnp.float32),
    )(h, p, p, cnt3, cnt3)

    p, _ = _sc_step_nocnt(h, src_p, dst_p)
    out = pl.pallas_call(
        _finalize_body,
        grid=(N // ROW_BLK,),
        in_specs=[pl.BlockSpec((ROW_BLK, D), lambda i: (i, 0)),
                  pl.BlockSpec((1, ROW_BLK, D), lambda i: (0, i, 0)),
                  pl.BlockSpec((1, ROW_BLK, D), lambda i: (1, i, 0)),
                  pl.BlockSpec((1, ROW_BLK, 1), lambda i: (0, i, 0)),
                  pl.BlockSpec((1, ROW_BLK, 1), lambda i: (1, i, 0)),
                  pl.BlockSpec((1, D), lambda i: (0, 0)),
                  pl.BlockSpec((1, 1), lambda i: (0, 0))],
        out_specs=pl.BlockSpec((1, 1), lambda i: (0, 0)),
        out_shape=jax.ShapeDtypeStruct((1, 1), jnp.float32),
        scratch_shapes=[pltpu.VMEM((1, D), jnp.float32)],
    )(h, p, p, cnt3, cnt3,
      W_pred[:D, 0].reshape(1, D), b_pred.reshape(1, 1))

    return out.reshape(1)
